# baseline (speedup 1.0000x reference)
import functools

import jax
import jax.numpy as jnp
from jax import lax
from jax.experimental import pallas as pl
from jax.experimental.pallas import tpu as pltpu
from jax.experimental.pallas import tpu_sc as plsc

_EPS = 1e-12
_L = 16


def _cbnorm_body(emb_ref, cbn_ref, b2_ref):
    e = emb_ref[...]
    n = jnp.sqrt(_rowsum(e * e))
    cbn = e / jnp.maximum(n, _EPS)
    cbn_ref[...] = cbn
    b2_ref[...] = _rowsum(cbn * cbn)


def _rowsum(x2):
    k = x2.shape[1]
    acc = x2[:, 0:128]
    for off in range(128, k, 128):
        acc = acc + x2[:, off:off + 128]
    acc_t = jnp.transpose(acc)
    u = acc_t[0:8, :]
    for j in range(1, 16):
        u = u + acc_t[8 * j:8 * j + 8, :]
    v = u[0:4, :] + u[4:8, :]
    w = v[0:2, :] + v[2:4, :]
    t = w[0:1, :] + w[1:2, :]
    return jnp.transpose(t)


def _dist_body(z_ref, cbn_ref, b_ref, idx_ref, zn_ref):
    zb = z_ref[...]
    n = jnp.sqrt(_rowsum(zb * zb))
    zn = zb / jnp.maximum(n, _EPS)
    a = _rowsum(zn * zn)
    m = lax.dot_general(zn, cbn_ref[...], (((1,), (0,)), ((), ())),
                        preferred_element_type=jnp.float32)
    d = a - 2.0 * m + b_ref[...]
    idx_ref[...] = jnp.argmin(d, axis=1).astype(jnp.int32)
    zn_ref[...] = zn


def _proj_body(zn_hbm, cbn_hbm, idx_hbm, out_hbm, idx_all,
               z0, z1, c0, c1, o0, o1, sem0, sem1, semo0, semo1,
               *, tpw, d, out_off=0):
    nc = 2
    wid = lax.axis_index("s") * nc + lax.axis_index("c")
    base = wid * tpw
    pltpu.sync_copy(idx_hbm.at[pl.ds(base, tpw)], idx_all)
    nj = d // _L
    nchunks = tpw // _L

    def start_in(g, zbuf, cbuf, sem):
        idxv = idx_all[pl.ds(g * _L, _L)]
        pltpu.async_copy(cbn_hbm.at[idxv], cbuf, sem)
        pltpu.async_copy(zn_hbm.at[pl.ds(base + g * _L, _L)], zbuf, sem)

    def wait_in(zbuf, cbuf, sem):
        idxv = idx_all[pl.ds(0, _L)]
        pltpu.make_async_copy(cbn_hbm.at[idxv], cbuf, sem).wait()
        pltpu.make_async_copy(zn_hbm.at[pl.ds(base, _L)], zbuf, sem).wait()

    def wait_out(obuf, sem):
        pltpu.make_async_copy(obuf, out_hbm.at[pl.ds(base, _L)], sem).wait()

    def compute(z_v, c_v, o_v):
        def token(t, carry2):
            a0 = jnp.zeros((_L,), jnp.float32)
            a1 = jnp.zeros((_L,), jnp.float32)
            a2 = jnp.zeros((_L,), jnp.float32)
            a3 = jnp.zeros((_L,), jnp.float32)
            for j in range(0, nj, 4):
                a0 = a0 + z_v[t, pl.ds(j * _L, _L)] * c_v[t, pl.ds(j * _L, _L)]
                a1 = a1 + z_v[t, pl.ds((j + 1) * _L, _L)] * c_v[t, pl.ds((j + 1) * _L, _L)]
                a2 = a2 + z_v[t, pl.ds((j + 2) * _L, _L)] * c_v[t, pl.ds((j + 2) * _L, _L)]
                a3 = a3 + z_v[t, pl.ds((j + 3) * _L, _L)] * c_v[t, pl.ds((j + 3) * _L, _L)]
            s = jnp.sum((a0 + a1) + (a2 + a3))
            for j in range(nj):
                o_v[t, pl.ds(j * _L, _L)] = s * c_v[t, pl.ds(j * _L, _L)]
            return carry2

        lax.fori_loop(0, _L, token, 0)

    start_in(0, z0, c0, sem0)
    start_in(1, z1, c1, sem1)

    def iter2(i, carry):
        g0 = 2 * i
        g1 = g0 + 1

        wait_in(z0, c0, sem0)

        @pl.when(i > 0)
        def _():
            wait_out(o0, semo0)

        compute(z0, c0, o0)
        pltpu.async_copy(o0, out_hbm.at[pl.ds(out_off + base + g0 * _L, _L)], semo0)

        @pl.when(g0 + 2 < nchunks)
        def _():
            start_in(g0 + 2, z0, c0, sem0)

        wait_in(z1, c1, sem1)

        @pl.when(i > 0)
        def _():
            wait_out(o1, semo1)

        compute(z1, c1, o1)
        pltpu.async_copy(o1, out_hbm.at[pl.ds(out_off + base + g1 * _L, _L)], semo1)

        @pl.when(g1 + 2 < nchunks)
        def _():
            start_in(g1 + 2, z1, c1, sem1)

        return carry

    lax.fori_loop(0, nchunks // 2, iter2, 0)
    wait_out(o0, semo0)
    wait_out(o1, semo1)


def kernel(z, embedding):
    n, dim = z.shape
    cdim = embedding.shape[0]

    cbn, b2 = pl.pallas_call(
        _cbnorm_body,
        out_shape=[jax.ShapeDtypeStruct((cdim, dim), jnp.float32),
                   jax.ShapeDtypeStruct((cdim, 1), jnp.float32)],
    )(embedding)
    b_row = b2.reshape(1, cdim)

    bm = 512
    nw = 32
    nchunks = 4
    cn = n // nchunks
    tpw = cn // nw

    out_ref = jax.new_ref(jnp.zeros((n, dim), jnp.float32))
    mesh = plsc.VectorSubcoreMesh(core_axis_name="c", subcore_axis_name="s")
    sc_scratch = [pltpu.VMEM((tpw,), jnp.int32),
                  pltpu.VMEM((_L, dim), jnp.float32),
                  pltpu.VMEM((_L, dim), jnp.float32),
                  pltpu.VMEM((_L, dim), jnp.float32),
                  pltpu.VMEM((_L, dim), jnp.float32),
                  pltpu.VMEM((_L, dim), jnp.float32),
                  pltpu.VMEM((_L, dim), jnp.float32),
                  pltpu.SemaphoreType.DMA,
                  pltpu.SemaphoreType.DMA,
                  pltpu.SemaphoreType.DMA,
                  pltpu.SemaphoreType.DMA]

    idx_chunks = []
    for c in range(nchunks):
        blk0 = c * (cn // bm)
        idx_c, zn_c = pl.pallas_call(
            _dist_body,
            grid=(cn // bm,),
            in_specs=[pl.BlockSpec((bm, dim), lambda i, b=blk0: (i + b, 0)),
                      pl.BlockSpec((cdim, dim), lambda i: (0, 0)),
                      pl.BlockSpec((1, cdim), lambda i: (0, 0))],
            out_specs=[pl.BlockSpec((bm,), lambda i: (i,)),
                       pl.BlockSpec((bm, dim), lambda i: (i, 0))],
            out_shape=[jax.ShapeDtypeStruct((cn,), jnp.int32),
                       jax.ShapeDtypeStruct((cn, dim), jnp.float32)],
        )(z, cbn, b_row)
        idx_chunks.append(idx_c)

        pl.kernel(
            functools.partial(_proj_body, tpw=tpw, d=dim, out_off=c * cn),
            out_type=(),
            mesh=mesh,
            compiler_params=pltpu.CompilerParams(needs_layout_passes=False),
            scratch_types=sc_scratch,
        )(zn_c, cbn, idx_c, out_ref)

    idx = jnp.concatenate(idx_chunks)
    return (out_ref[...], idx)


# speedup vs baseline: 1.1173x; 1.1173x over previous
import functools

import jax
import jax.numpy as jnp
from jax import lax
from jax.experimental import pallas as pl
from jax.experimental.pallas import tpu as pltpu
from jax.experimental.pallas import tpu_sc as plsc

_EPS = 1e-12
_L = 16


def _cbnorm_body(emb_ref, cbn_ref, b2_ref):
    e = emb_ref[...]
    n = jnp.sqrt(_rowsum(e * e))
    cbn = e / jnp.maximum(n, _EPS)
    cbn_ref[...] = cbn
    b2_ref[...] = _rowsum(cbn * cbn)


def _rowsum(x2):
    k = x2.shape[1]
    acc = x2[:, 0:128]
    for off in range(128, k, 128):
        acc = acc + x2[:, off:off + 128]
    acc_t = jnp.transpose(acc)
    u = acc_t[0:8, :]
    for j in range(1, 16):
        u = u + acc_t[8 * j:8 * j + 8, :]
    v = u[0:4, :] + u[4:8, :]
    w = v[0:2, :] + v[2:4, :]
    t = w[0:1, :] + w[1:2, :]
    return jnp.transpose(t)


def _dist_body(z_ref, cbn_ref, b_ref, idx_ref, zn_ref):
    zb = z_ref[...]
    n = jnp.sqrt(_rowsum(zb * zb))
    zn = zb / jnp.maximum(n, _EPS)
    a = _rowsum(zn * zn)
    m = lax.dot_general(zn, cbn_ref[...], (((1,), (0,)), ((), ())),
                        preferred_element_type=jnp.float32)
    d = a - 2.0 * m + b_ref[...]
    idx_ref[...] = jnp.argmin(d, axis=1).astype(jnp.int32)
    zn_ref[...] = zn


def _proj_body(zn_hbm, cbn_hbm, idx_hbm, out_hbm, idx_all,
               z0, z1, c0, c1, o0, o1, sem0, sem1, semo0, semo1,
               *, tpw, d, out_off=0):
    nc = 2
    wid = lax.axis_index("s") * nc + lax.axis_index("c")
    base = wid * tpw
    pltpu.sync_copy(idx_hbm.at[pl.ds(base, tpw)], idx_all)
    nj = d // _L
    nchunks = tpw // _L

    def start_in(g, zbuf, cbuf, sem):
        idxv = idx_all[pl.ds(g * _L, _L)]
        pltpu.async_copy(cbn_hbm.at[idxv], cbuf, sem)
        pltpu.async_copy(zn_hbm.at[pl.ds(base + g * _L, _L)], zbuf, sem)

    def wait_in(zbuf, cbuf, sem):
        idxv = idx_all[pl.ds(0, _L)]
        pltpu.make_async_copy(cbn_hbm.at[idxv], cbuf, sem).wait()
        pltpu.make_async_copy(zn_hbm.at[pl.ds(base, _L)], zbuf, sem).wait()

    def wait_out(obuf, sem):
        pltpu.make_async_copy(obuf, out_hbm.at[pl.ds(base, _L)], sem).wait()

    def compute(z_v, c_v, o_v):
        def token(t, carry2):
            a0 = jnp.zeros((_L,), jnp.float32)
            a1 = jnp.zeros((_L,), jnp.float32)
            a2 = jnp.zeros((_L,), jnp.float32)
            a3 = jnp.zeros((_L,), jnp.float32)
            for j in range(0, nj, 4):
                a0 = a0 + z_v[t, pl.ds(j * _L, _L)] * c_v[t, pl.ds(j * _L, _L)]
                a1 = a1 + z_v[t, pl.ds((j + 1) * _L, _L)] * c_v[t, pl.ds((j + 1) * _L, _L)]
                a2 = a2 + z_v[t, pl.ds((j + 2) * _L, _L)] * c_v[t, pl.ds((j + 2) * _L, _L)]
                a3 = a3 + z_v[t, pl.ds((j + 3) * _L, _L)] * c_v[t, pl.ds((j + 3) * _L, _L)]
            s = jnp.sum((a0 + a1) + (a2 + a3))
            for j in range(nj):
                o_v[t, pl.ds(j * _L, _L)] = s * c_v[t, pl.ds(j * _L, _L)]
            return carry2

        lax.fori_loop(0, _L, token, 0)

    start_in(0, z0, c0, sem0)
    start_in(1, z1, c1, sem1)

    def iter2(i, carry):
        g0 = 2 * i
        g1 = g0 + 1

        wait_in(z0, c0, sem0)

        @pl.when(i > 0)
        def _():
            wait_out(o0, semo0)

        compute(z0, c0, o0)
        pltpu.async_copy(o0, out_hbm.at[pl.ds(out_off + base + g0 * _L, _L)], semo0)

        @pl.when(g0 + 2 < nchunks)
        def _():
            start_in(g0 + 2, z0, c0, sem0)

        wait_in(z1, c1, sem1)

        @pl.when(i > 0)
        def _():
            wait_out(o1, semo1)

        compute(z1, c1, o1)
        pltpu.async_copy(o1, out_hbm.at[pl.ds(out_off + base + g1 * _L, _L)], semo1)

        @pl.when(g1 + 2 < nchunks)
        def _():
            start_in(g1 + 2, z1, c1, sem1)

        return carry

    lax.fori_loop(0, nchunks // 2, iter2, 0)
    wait_out(o0, semo0)
    wait_out(o1, semo1)


def kernel(z, embedding):
    n, dim = z.shape
    cdim = embedding.shape[0]

    cbn, b2 = pl.pallas_call(
        _cbnorm_body,
        out_shape=[jax.ShapeDtypeStruct((cdim, dim), jnp.float32),
                   jax.ShapeDtypeStruct((cdim, 1), jnp.float32)],
    )(embedding)
    b_row = b2.reshape(1, cdim)

    bm = 512
    nw = 32
    nchunks = 4
    cn = n // nchunks
    tpw = cn // nw

    out_ref = jax.new_ref(jax.lax.empty((n, dim), jnp.float32))
    mesh = plsc.VectorSubcoreMesh(core_axis_name="c", subcore_axis_name="s")
    sc_scratch = [pltpu.VMEM((tpw,), jnp.int32),
                  pltpu.VMEM((_L, dim), jnp.float32),
                  pltpu.VMEM((_L, dim), jnp.float32),
                  pltpu.VMEM((_L, dim), jnp.float32),
                  pltpu.VMEM((_L, dim), jnp.float32),
                  pltpu.VMEM((_L, dim), jnp.float32),
                  pltpu.VMEM((_L, dim), jnp.float32),
                  pltpu.SemaphoreType.DMA,
                  pltpu.SemaphoreType.DMA,
                  pltpu.SemaphoreType.DMA,
                  pltpu.SemaphoreType.DMA]

    idx_chunks = []
    for c in range(nchunks):
        blk0 = c * (cn // bm)
        idx_c, zn_c = pl.pallas_call(
            _dist_body,
            grid=(cn // bm,),
            in_specs=[pl.BlockSpec((bm, dim), lambda i, b=blk0: (i + b, 0)),
                      pl.BlockSpec((cdim, dim), lambda i: (0, 0)),
                      pl.BlockSpec((1, cdim), lambda i: (0, 0))],
            out_specs=[pl.BlockSpec((bm,), lambda i: (i,)),
                       pl.BlockSpec((bm, dim), lambda i: (i, 0))],
            out_shape=[jax.ShapeDtypeStruct((cn,), jnp.int32),
                       jax.ShapeDtypeStruct((cn, dim), jnp.float32)],
        )(z, cbn, b_row)
        idx_chunks.append(idx_c)

        pl.kernel(
            functools.partial(_proj_body, tpw=tpw, d=dim, out_off=c * cn),
            out_type=(),
            mesh=mesh,
            compiler_params=pltpu.CompilerParams(needs_layout_passes=False),
            scratch_types=sc_scratch,
        )(zn_c, cbn, idx_c, out_ref)

    idx = jnp.concatenate(idx_chunks)
    return (out_ref[...], idx)


# speedup vs baseline: 1.1457x; 1.0255x over previous
import functools

import jax
import jax.numpy as jnp
from jax import lax
from jax.experimental import pallas as pl
from jax.experimental.pallas import tpu as pltpu
from jax.experimental.pallas import tpu_sc as plsc

_EPS = 1e-12
_L = 16


def _cbnorm_body(emb_ref, cbn_ref, b2_ref):
    e = emb_ref[...]
    n = jnp.sqrt(_rowsum(e * e))
    cbn = e / jnp.maximum(n, _EPS)
    cbn_ref[...] = cbn
    b2_ref[...] = _rowsum(cbn * cbn)


def _rowsum(x2):
    k = x2.shape[1]
    acc = x2[:, 0:128]
    for off in range(128, k, 128):
        acc = acc + x2[:, off:off + 128]
    acc_t = jnp.transpose(acc)
    u = acc_t[0:8, :]
    for j in range(1, 16):
        u = u + acc_t[8 * j:8 * j + 8, :]
    v = u[0:4, :] + u[4:8, :]
    w = v[0:2, :] + v[2:4, :]
    t = w[0:1, :] + w[1:2, :]
    return jnp.transpose(t)


def _dist_body(z_ref, cbn_ref, b_ref, idx_ref, zn_ref):
    zb = z_ref[...]
    n = jnp.sqrt(_rowsum(zb * zb))
    zn = zb / jnp.maximum(n, _EPS)
    a = _rowsum(zn * zn)
    m = lax.dot_general(zn, cbn_ref[...], (((1,), (0,)), ((), ())),
                        preferred_element_type=jnp.float32)
    d = a - 2.0 * m + b_ref[...]
    idx_ref[...] = jnp.argmin(d, axis=1).astype(jnp.int32)
    zn_ref[...] = zn


def _proj_body(zn_hbm, cbn_hbm, idx_hbm, out_hbm, idx_all,
               z0, z1, c0, c1, o0, o1, sem0, sem1, semo0, semo1,
               *, tpw, d, out_off=0):
    nc = 2
    wid = lax.axis_index("s") * nc + lax.axis_index("c")
    base = wid * tpw
    pltpu.sync_copy(idx_hbm.at[pl.ds(base, tpw)], idx_all)
    nj = d // _L
    nchunks = tpw // _L

    def start_in(g, zbuf, cbuf, sem):
        idxv = idx_all[pl.ds(g * _L, _L)]
        pltpu.async_copy(cbn_hbm.at[idxv], cbuf, sem)
        pltpu.async_copy(zn_hbm.at[pl.ds(base + g * _L, _L)], zbuf, sem)

    def wait_in(zbuf, cbuf, sem):
        idxv = idx_all[pl.ds(0, _L)]
        pltpu.make_async_copy(cbn_hbm.at[idxv], cbuf, sem).wait()
        pltpu.make_async_copy(zn_hbm.at[pl.ds(base, _L)], zbuf, sem).wait()

    def wait_out(obuf, sem):
        pltpu.make_async_copy(obuf, out_hbm.at[pl.ds(base, _L)], sem).wait()

    def compute(z_v, c_v, o_v):
        def token(t, carry2):
            a0 = jnp.zeros((_L,), jnp.float32)
            a1 = jnp.zeros((_L,), jnp.float32)
            a2 = jnp.zeros((_L,), jnp.float32)
            a3 = jnp.zeros((_L,), jnp.float32)
            for j in range(0, nj, 4):
                a0 = a0 + z_v[t, pl.ds(j * _L, _L)] * c_v[t, pl.ds(j * _L, _L)]
                a1 = a1 + z_v[t, pl.ds((j + 1) * _L, _L)] * c_v[t, pl.ds((j + 1) * _L, _L)]
                a2 = a2 + z_v[t, pl.ds((j + 2) * _L, _L)] * c_v[t, pl.ds((j + 2) * _L, _L)]
                a3 = a3 + z_v[t, pl.ds((j + 3) * _L, _L)] * c_v[t, pl.ds((j + 3) * _L, _L)]
            s = jnp.sum((a0 + a1) + (a2 + a3))
            for j in range(nj):
                o_v[t, pl.ds(j * _L, _L)] = s * c_v[t, pl.ds(j * _L, _L)]
            return carry2

        lax.fori_loop(0, _L, token, 0)

    start_in(0, z0, c0, sem0)
    start_in(1, z1, c1, sem1)

    def iter2(i, carry):
        g0 = 2 * i
        g1 = g0 + 1

        wait_in(z0, c0, sem0)

        @pl.when(i > 0)
        def _():
            wait_out(o0, semo0)

        compute(z0, c0, o0)
        pltpu.async_copy(o0, out_hbm.at[pl.ds(out_off + base + g0 * _L, _L)], semo0)

        @pl.when(g0 + 2 < nchunks)
        def _():
            start_in(g0 + 2, z0, c0, sem0)

        wait_in(z1, c1, sem1)

        @pl.when(i > 0)
        def _():
            wait_out(o1, semo1)

        compute(z1, c1, o1)
        pltpu.async_copy(o1, out_hbm.at[pl.ds(out_off + base + g1 * _L, _L)], semo1)

        @pl.when(g1 + 2 < nchunks)
        def _():
            start_in(g1 + 2, z1, c1, sem1)

        return carry

    lax.fori_loop(0, nchunks // 2, iter2, 0)
    wait_out(o0, semo0)
    wait_out(o1, semo1)


def kernel(z, embedding):
    n, dim = z.shape
    cdim = embedding.shape[0]

    cbn, b2 = pl.pallas_call(
        _cbnorm_body,
        out_shape=[jax.ShapeDtypeStruct((cdim, dim), jnp.float32),
                   jax.ShapeDtypeStruct((cdim, 1), jnp.float32)],
    )(embedding)
    b_row = b2.reshape(1, cdim)

    bm = 512
    nw = 32
    nchunks = 2
    cn = n // nchunks
    tpw = cn // nw

    out_ref = jax.new_ref(jax.lax.empty((n, dim), jnp.float32))
    mesh = plsc.VectorSubcoreMesh(core_axis_name="c", subcore_axis_name="s")
    sc_scratch = [pltpu.VMEM((tpw,), jnp.int32),
                  pltpu.VMEM((_L, dim), jnp.float32),
                  pltpu.VMEM((_L, dim), jnp.float32),
                  pltpu.VMEM((_L, dim), jnp.float32),
                  pltpu.VMEM((_L, dim), jnp.float32),
                  pltpu.VMEM((_L, dim), jnp.float32),
                  pltpu.VMEM((_L, dim), jnp.float32),
                  pltpu.SemaphoreType.DMA,
                  pltpu.SemaphoreType.DMA,
                  pltpu.SemaphoreType.DMA,
                  pltpu.SemaphoreType.DMA]

    idx_chunks = []
    for c in range(nchunks):
        blk0 = c * (cn // bm)
        idx_c, zn_c = pl.pallas_call(
            _dist_body,
            grid=(cn // bm,),
            in_specs=[pl.BlockSpec((bm, dim), lambda i, b=blk0: (i + b, 0)),
                      pl.BlockSpec((cdim, dim), lambda i: (0, 0)),
                      pl.BlockSpec((1, cdim), lambda i: (0, 0))],
            out_specs=[pl.BlockSpec((bm,), lambda i: (i,)),
                       pl.BlockSpec((bm, dim), lambda i: (i, 0))],
            out_shape=[jax.ShapeDtypeStruct((cn,), jnp.int32),
                       jax.ShapeDtypeStruct((cn, dim), jnp.float32)],
        )(z, cbn, b_row)
        idx_chunks.append(idx_c)

        pl.kernel(
            functools.partial(_proj_body, tpw=tpw, d=dim, out_off=c * cn),
            out_type=(),
            mesh=mesh,
            compiler_params=pltpu.CompilerParams(needs_layout_passes=False),
            scratch_types=sc_scratch,
        )(zn_c, cbn, idx_c, out_ref)

    idx = jnp.concatenate(idx_chunks)
    return (out_ref[...], idx)


# speedup vs baseline: 1.1539x; 1.0071x over previous
import functools

import jax
import jax.numpy as jnp
from jax import lax
from jax.experimental import pallas as pl
from jax.experimental.pallas import tpu as pltpu
from jax.experimental.pallas import tpu_sc as plsc

_EPS = 1e-12
_L = 16


def _cbnorm_body(emb_ref, cbn_ref, b2_ref):
    e = emb_ref[...]
    n = jnp.sqrt(_rowsum(e * e))
    cbn = e / jnp.maximum(n, _EPS)
    cbn_ref[...] = cbn
    b2_ref[...] = _rowsum(cbn * cbn)


def _rowsum(x2):
    k = x2.shape[1]
    acc = x2[:, 0:128]
    for off in range(128, k, 128):
        acc = acc + x2[:, off:off + 128]
    acc_t = jnp.transpose(acc)
    u = acc_t[0:8, :]
    for j in range(1, 16):
        u = u + acc_t[8 * j:8 * j + 8, :]
    v = u[0:4, :] + u[4:8, :]
    w = v[0:2, :] + v[2:4, :]
    t = w[0:1, :] + w[1:2, :]
    return jnp.transpose(t)


def _dist_body(z_ref, cbn_ref, b_ref, idx_ref, zn_ref):
    zb = z_ref[...]
    n = jnp.sqrt(_rowsum(zb * zb))
    zn = zb / jnp.maximum(n, _EPS)
    a = _rowsum(zn * zn)
    m = lax.dot_general(zn, cbn_ref[...], (((1,), (0,)), ((), ())),
                        preferred_element_type=jnp.float32)
    d = a - 2.0 * m + b_ref[...]
    idx_ref[...] = jnp.argmin(d, axis=1).astype(jnp.int32)
    zn_ref[...] = zn


def _proj_body(zn_hbm, cbn_hbm, idx_hbm, out_hbm, idx_all,
               z0, z1, c0, c1, o0, o1, sem0, sem1, semo0, semo1,
               *, tpw, d, out_off=0):
    nc = 2
    wid = lax.axis_index("s") * nc + lax.axis_index("c")
    base = wid * tpw
    pltpu.sync_copy(idx_hbm.at[pl.ds(base, tpw)], idx_all)
    nj = d // _L
    nchunks = tpw // _L

    def start_in(g, zbuf, cbuf, sem):
        idxv = idx_all[pl.ds(g * _L, _L)]
        pltpu.async_copy(cbn_hbm.at[idxv], cbuf, sem)
        pltpu.async_copy(zn_hbm.at[pl.ds(base + g * _L, _L)], zbuf, sem)

    def wait_in(zbuf, cbuf, sem):
        idxv = idx_all[pl.ds(0, _L)]
        pltpu.make_async_copy(cbn_hbm.at[idxv], cbuf, sem).wait()
        pltpu.make_async_copy(zn_hbm.at[pl.ds(base, _L)], zbuf, sem).wait()

    def wait_out(obuf, sem):
        pltpu.make_async_copy(obuf, out_hbm.at[pl.ds(base, _L)], sem).wait()

    def compute(z_v, c_v, o_v):
        def token(t, carry2):
            a0 = jnp.zeros((_L,), jnp.float32)
            a1 = jnp.zeros((_L,), jnp.float32)
            a2 = jnp.zeros((_L,), jnp.float32)
            a3 = jnp.zeros((_L,), jnp.float32)
            for j in range(0, nj, 4):
                a0 = a0 + z_v[t, pl.ds(j * _L, _L)] * c_v[t, pl.ds(j * _L, _L)]
                a1 = a1 + z_v[t, pl.ds((j + 1) * _L, _L)] * c_v[t, pl.ds((j + 1) * _L, _L)]
                a2 = a2 + z_v[t, pl.ds((j + 2) * _L, _L)] * c_v[t, pl.ds((j + 2) * _L, _L)]
                a3 = a3 + z_v[t, pl.ds((j + 3) * _L, _L)] * c_v[t, pl.ds((j + 3) * _L, _L)]
            s = jnp.sum((a0 + a1) + (a2 + a3))
            for j in range(nj):
                o_v[t, pl.ds(j * _L, _L)] = s * c_v[t, pl.ds(j * _L, _L)]
            return carry2

        lax.fori_loop(0, _L, token, 0)

    start_in(0, z0, c0, sem0)
    start_in(1, z1, c1, sem1)

    def iter2(i, carry):
        g0 = 2 * i
        g1 = g0 + 1

        wait_in(z0, c0, sem0)

        @pl.when(i > 0)
        def _():
            wait_out(o0, semo0)

        compute(z0, c0, o0)
        pltpu.async_copy(o0, out_hbm.at[pl.ds(out_off + base + g0 * _L, _L)], semo0)

        @pl.when(g0 + 2 < nchunks)
        def _():
            start_in(g0 + 2, z0, c0, sem0)

        wait_in(z1, c1, sem1)

        @pl.when(i > 0)
        def _():
            wait_out(o1, semo1)

        compute(z1, c1, o1)
        pltpu.async_copy(o1, out_hbm.at[pl.ds(out_off + base + g1 * _L, _L)], semo1)

        @pl.when(g1 + 2 < nchunks)
        def _():
            start_in(g1 + 2, z1, c1, sem1)

        return carry

    lax.fori_loop(0, nchunks // 2, iter2, 0)
    wait_out(o0, semo0)
    wait_out(o1, semo1)


def kernel(z, embedding):
    n, dim = z.shape
    cdim = embedding.shape[0]

    cbn, b2 = pl.pallas_call(
        _cbnorm_body,
        out_shape=[jax.ShapeDtypeStruct((cdim, dim), jnp.float32),
                   jax.ShapeDtypeStruct((cdim, 1), jnp.float32)],
    )(embedding)
    b_row = b2.reshape(1, cdim)

    bm = 512
    nw = 32
    sizes = (2048, 7168, 7168)

    out_ref = jax.new_ref(jax.lax.empty((n, dim), jnp.float32))
    mesh = plsc.VectorSubcoreMesh(core_axis_name="c", subcore_axis_name="s")

    def sc_scratch(tpw):
        return [pltpu.VMEM((tpw,), jnp.int32),
                pltpu.VMEM((_L, dim), jnp.float32),
                pltpu.VMEM((_L, dim), jnp.float32),
                pltpu.VMEM((_L, dim), jnp.float32),
                pltpu.VMEM((_L, dim), jnp.float32),
                pltpu.VMEM((_L, dim), jnp.float32),
                pltpu.VMEM((_L, dim), jnp.float32),
                pltpu.SemaphoreType.DMA,
                pltpu.SemaphoreType.DMA,
                pltpu.SemaphoreType.DMA,
                pltpu.SemaphoreType.DMA]

    idx_chunks = []
    row0 = 0
    for cn in sizes:
        blk0 = row0 // bm
        tpw = cn // nw
        idx_c, zn_c = pl.pallas_call(
            _dist_body,
            grid=(cn // bm,),
            in_specs=[pl.BlockSpec((bm, dim), lambda i, b=blk0: (i + b, 0)),
                      pl.BlockSpec((cdim, dim), lambda i: (0, 0)),
                      pl.BlockSpec((1, cdim), lambda i: (0, 0))],
            out_specs=[pl.BlockSpec((bm,), lambda i: (i,)),
                       pl.BlockSpec((bm, dim), lambda i: (i, 0))],
            out_shape=[jax.ShapeDtypeStruct((cn,), jnp.int32),
                       jax.ShapeDtypeStruct((cn, dim), jnp.float32)],
        )(z, cbn, b_row)
        idx_chunks.append(idx_c)

        pl.kernel(
            functools.partial(_proj_body, tpw=tpw, d=dim, out_off=row0),
            out_type=(),
            mesh=mesh,
            compiler_params=pltpu.CompilerParams(needs_layout_passes=False),
            scratch_types=sc_scratch(tpw),
        )(zn_c, cbn, idx_c, out_ref)
        row0 += cn

    idx = jnp.concatenate(idx_chunks)
    return (out_ref[...], idx)


# speedup vs baseline: 1.1671x; 1.0115x over previous
import functools

import jax
import jax.numpy as jnp
from jax import lax
from jax.experimental import pallas as pl
from jax.experimental.pallas import tpu as pltpu
from jax.experimental.pallas import tpu_sc as plsc

_EPS = 1e-12
_L = 16


def _cbnorm_body(emb_ref, cbn_ref, b2_ref):
    e = emb_ref[...]
    n = jnp.sqrt(_rowsum(e * e))
    cbn = e / jnp.maximum(n, _EPS)
    cbn_ref[...] = cbn
    b2_ref[...] = _rowsum(cbn * cbn)


def _rowsum(x2):
    k = x2.shape[1]
    acc = x2[:, 0:128]
    for off in range(128, k, 128):
        acc = acc + x2[:, off:off + 128]
    acc_t = jnp.transpose(acc)
    u = acc_t[0:8, :]
    for j in range(1, 16):
        u = u + acc_t[8 * j:8 * j + 8, :]
    v = u[0:4, :] + u[4:8, :]
    w = v[0:2, :] + v[2:4, :]
    t = w[0:1, :] + w[1:2, :]
    return jnp.transpose(t)


def _dist_body(z_ref, cbn_ref, b_ref, idx_ref, zn_ref):
    zb = z_ref[...]
    n = jnp.sqrt(_rowsum(zb * zb))
    zn = zb / jnp.maximum(n, _EPS)
    a = _rowsum(zn * zn)
    m = lax.dot_general(zn, cbn_ref[...], (((1,), (0,)), ((), ())),
                        preferred_element_type=jnp.float32)
    d = a - 2.0 * m + b_ref[...]
    idx_ref[...] = jnp.argmin(d, axis=1).astype(jnp.int32)
    zn_ref[...] = zn


def _proj_body(zn_hbm, cbn_hbm, idx_hbm, out_hbm, idx_all,
               z0, z1, c0, c1, o0, o1, sem0, sem1, semo0, semo1,
               *, tpw, d, out_off=0):
    nc = 2
    wid = lax.axis_index("s") * nc + lax.axis_index("c")
    base = wid * tpw
    pltpu.sync_copy(idx_hbm.at[pl.ds(base, tpw)], idx_all)
    nj = d // _L
    nchunks = tpw // _L

    def start_in(g, zbuf, cbuf, sem):
        idxv = idx_all[pl.ds(g * _L, _L)]
        pltpu.async_copy(cbn_hbm.at[idxv], cbuf, sem)
        pltpu.async_copy(zn_hbm.at[pl.ds(base + g * _L, _L)], zbuf, sem)

    def wait_in(zbuf, cbuf, sem):
        idxv = idx_all[pl.ds(0, _L)]
        pltpu.make_async_copy(cbn_hbm.at[idxv], cbuf, sem).wait()
        pltpu.make_async_copy(zn_hbm.at[pl.ds(base, _L)], zbuf, sem).wait()

    def wait_out(obuf, sem):
        pltpu.make_async_copy(obuf, out_hbm.at[pl.ds(base, _L)], sem).wait()

    def compute(z_v, c_v, o_v):
        def token(t, carry2):
            a0 = jnp.zeros((_L,), jnp.float32)
            a1 = jnp.zeros((_L,), jnp.float32)
            a2 = jnp.zeros((_L,), jnp.float32)
            a3 = jnp.zeros((_L,), jnp.float32)
            for j in range(0, nj, 4):
                a0 = a0 + z_v[t, pl.ds(j * _L, _L)] * c_v[t, pl.ds(j * _L, _L)]
                a1 = a1 + z_v[t, pl.ds((j + 1) * _L, _L)] * c_v[t, pl.ds((j + 1) * _L, _L)]
                a2 = a2 + z_v[t, pl.ds((j + 2) * _L, _L)] * c_v[t, pl.ds((j + 2) * _L, _L)]
                a3 = a3 + z_v[t, pl.ds((j + 3) * _L, _L)] * c_v[t, pl.ds((j + 3) * _L, _L)]
            s = jnp.sum((a0 + a1) + (a2 + a3))
            for j in range(nj):
                o_v[t, pl.ds(j * _L, _L)] = s * c_v[t, pl.ds(j * _L, _L)]
            return carry2

        lax.fori_loop(0, _L, token, 0)

    start_in(0, z0, c0, sem0)
    start_in(1, z1, c1, sem1)

    def iter2(i, carry):
        g0 = 2 * i
        g1 = g0 + 1

        wait_in(z0, c0, sem0)

        @pl.when(i > 0)
        def _():
            wait_out(o0, semo0)

        compute(z0, c0, o0)
        pltpu.async_copy(o0, out_hbm.at[pl.ds(out_off + base + g0 * _L, _L)], semo0)

        @pl.when(g0 + 2 < nchunks)
        def _():
            start_in(g0 + 2, z0, c0, sem0)

        wait_in(z1, c1, sem1)

        @pl.when(i > 0)
        def _():
            wait_out(o1, semo1)

        compute(z1, c1, o1)
        pltpu.async_copy(o1, out_hbm.at[pl.ds(out_off + base + g1 * _L, _L)], semo1)

        @pl.when(g1 + 2 < nchunks)
        def _():
            start_in(g1 + 2, z1, c1, sem1)

        return carry

    lax.fori_loop(0, nchunks // 2, iter2, 0)
    wait_out(o0, semo0)
    wait_out(o1, semo1)


def kernel(z, embedding):
    n, dim = z.shape
    cdim = embedding.shape[0]

    cbn, b2 = pl.pallas_call(
        _cbnorm_body,
        out_shape=[jax.ShapeDtypeStruct((cdim, dim), jnp.float32),
                   jax.ShapeDtypeStruct((cdim, 1), jnp.float32)],
    )(embedding)
    b_row = b2.reshape(1, cdim)

    bm = 512
    nw = 32
    sizes = (2048, 4096, 5120, 5120)

    out_ref = jax.new_ref(jax.lax.empty((n, dim), jnp.float32))
    mesh = plsc.VectorSubcoreMesh(core_axis_name="c", subcore_axis_name="s")

    def sc_scratch(tpw):
        return [pltpu.VMEM((tpw,), jnp.int32),
                pltpu.VMEM((_L, dim), jnp.float32),
                pltpu.VMEM((_L, dim), jnp.float32),
                pltpu.VMEM((_L, dim), jnp.float32),
                pltpu.VMEM((_L, dim), jnp.float32),
                pltpu.VMEM((_L, dim), jnp.float32),
                pltpu.VMEM((_L, dim), jnp.float32),
                pltpu.SemaphoreType.DMA,
                pltpu.SemaphoreType.DMA,
                pltpu.SemaphoreType.DMA,
                pltpu.SemaphoreType.DMA]

    idx_chunks = []
    row0 = 0
    for cn in sizes:
        blk0 = row0 // bm
        tpw = cn // nw
        idx_c, zn_c = pl.pallas_call(
            _dist_body,
            grid=(cn // bm,),
            in_specs=[pl.BlockSpec((bm, dim), lambda i, b=blk0: (i + b, 0)),
                      pl.BlockSpec((cdim, dim), lambda i: (0, 0)),
                      pl.BlockSpec((1, cdim), lambda i: (0, 0))],
            out_specs=[pl.BlockSpec((bm,), lambda i: (i,)),
                       pl.BlockSpec((bm, dim), lambda i: (i, 0))],
            out_shape=[jax.ShapeDtypeStruct((cn,), jnp.int32),
                       jax.ShapeDtypeStruct((cn, dim), jnp.float32)],
        )(z, cbn, b_row)
        idx_chunks.append(idx_c)

        pl.kernel(
            functools.partial(_proj_body, tpw=tpw, d=dim, out_off=row0),
            out_type=(),
            mesh=mesh,
            compiler_params=pltpu.CompilerParams(needs_layout_passes=False),
            scratch_types=sc_scratch(tpw),
        )(zn_c, cbn, idx_c, out_ref)
        row0 += cn

    idx = jnp.concatenate(idx_chunks)
    return (out_ref[...], idx)
